# Initial kernel scaffold; baseline (speedup 1.0000x reference)
import jax
import jax.numpy as jnp
from jax.experimental import pallas as pl


def kernel(x, edge_attr, edge_time, memory, W_time, b_time, W_i, W_h, b_i, b_h, a_src, a_dst, W_v, edge_index):
    raise NotImplementedError("write your pallas kernel here")


# speedup vs baseline: 5.3473x; 5.3473x over previous
import dataclasses
import functools

import jax
import jax.numpy as jnp
from jax import lax
from jax.experimental import pallas as pl
from jax.experimental.pallas import tpu as pltpu
from jax.experimental.pallas import tpu_sc as plsc

N = 10000
E = 320000
D = 128
DE = 16
NP = 10240
NW = 32
E16 = E // 16
EW = E // NW
CHA = 80
CHT = 200
NBT = 40
KBT = CHT // NBT
NCHT = EW // CHT
CHS = 400
NCHS = EW // CHS
NCH16 = E16 // CHA
RPT = NP // 16
ZR = 40
BT = 640
R = 1280

_mesh = plsc.VectorSubcoreMesh(core_axis_name="c", subcore_axis_name="s")

_cp = pltpu.CompilerParams()
if "needs_layout_passes" in pltpu.CompilerParams.__dataclass_fields__:
    _cp = dataclasses.replace(_cp, needs_layout_passes=False)


def _zero2d(ref, nrows, ncols):
    @pl.loop(0, nrows)
    def _(r):
        for t in range(ncols // 16):
            ref[r, pl.ds(t * 16, 16)] = jnp.zeros((16,), jnp.float32)


def _zero_acc(src_rows, acc, base):
    @pl.loop(0, RPT // ZR)
    def _(i):
        pltpu.sync_copy(src_rows.at[pl.ds(0, ZR)],
                        acc.at[pl.ds(base + i * ZR, ZR)])


@functools.partial(
    pl.kernel,
    mesh=_mesh,
    compiler_params=_cp,
    out_type=jax.ShapeDtypeStruct((2, NP, D), jnp.float32),
    scratch_types=[
        pltpu.VMEM((1, CHA), jnp.int32),
        pltpu.VMEM((1, CHA), jnp.int32),
        pltpu.VMEM((CHA, D), jnp.float32),
        pltpu.VMEM((CHA, DE), jnp.float32),
        pltpu.VMEM_SHARED((NP, D), jnp.float32),
        pltpu.SemaphoreType.DMA,
    ],
)
def _sc_a1(mem_hbm, ea_hbm, src_hbm, dst_hbm, out, sidx, didx, rows, earows,
           acc, sem):
    c = lax.axis_index("c")
    s = lax.axis_index("s")
    base = s * RPT
    one0 = jnp.where(lax.iota(jnp.int32, 16) == 0, 1.0, 0.0).astype(jnp.float32)

    _zero2d(rows, CHA, D)
    _zero_acc(rows, acc, base)
    plsc.subcore_barrier()

    @pl.loop(0, NCH16)
    def _(k):
        e0 = s * E16 + k * CHA
        pltpu.sync_copy(dst_hbm.at[pl.ds(e0, CHA)], didx.at[0])

        @pl.when(c == 0)
        def _():
            pltpu.sync_copy(src_hbm.at[pl.ds(e0, CHA)], sidx.at[0])
            pltpu.async_copy(mem_hbm.at[sidx.at[0]], rows, sem).wait()

        @pl.when(c == 1)
        def _():
            pltpu.sync_copy(ea_hbm.at[pl.ds(e0, CHA)], earows)

            @pl.loop(0, CHA)
            def _(r):
                rows[r, pl.ds(0, 16)] = earows[r, pl.ds(0, 16)]
                rows[r, pl.ds(16, 16)] = one0

        pltpu.sync_copy(rows, acc.at[didx.at[0]], add=True)

    plsc.subcore_barrier()
    pltpu.sync_copy(acc.at[pl.ds(base, RPT)], out.at[c, pl.ds(base, RPT)])


@functools.partial(
    pl.kernel,
    mesh=_mesh,
    out_type=jax.ShapeDtypeStruct((2, NP, D), jnp.float32),
    scratch_types=[
        pltpu.VMEM((KBT, NBT), jnp.int32),
        pltpu.VMEM((CHT, D), jnp.float32),
        pltpu.VMEM_SHARED((NP, D), jnp.float32),
        pltpu.SemaphoreType.DMA,
    ],
)
def _sc_a2(te_hbm, dst_hbm, ste_out, didx, rows, acct, sem):
    c = lax.axis_index("c")
    s = lax.axis_index("s")
    w = c * 16 + s
    base = s * RPT

    _zero2d(rows, ZR, D)
    _zero_acc(rows, acct, base)
    plsc.subcore_barrier()

    @pl.loop(0, NCHT)
    def _(k):
        e0 = w * EW + k * CHT
        for j in range(KBT):
            pltpu.sync_copy(dst_hbm.at[pl.ds(e0 + j * NBT, NBT)], didx.at[j])
        pltpu.sync_copy(te_hbm.at[pl.ds(e0, CHT)], rows)
        for j in range(KBT):
            pltpu.sync_copy(rows.at[pl.ds(j * NBT, NBT)],
                            acct.at[didx.at[j]], add=True)

    plsc.subcore_barrier()
    pltpu.sync_copy(acct.at[pl.ds(base, RPT)], ste_out.at[c, pl.ds(base, RPT)])


@functools.partial(
    pl.kernel,
    mesh=_mesh,
    compiler_params=_cp,
    out_type=(
        jax.ShapeDtypeStruct((E,), jnp.float32),
        jax.ShapeDtypeStruct((NW * 16,), jnp.float32),
    ),
    scratch_types=[
        pltpu.VMEM((NP,), jnp.float32),
        pltpu.VMEM((NP,), jnp.float32),
        pltpu.VMEM((CHS,), jnp.int32),
        pltpu.VMEM((CHS,), jnp.int32),
        pltpu.VMEM((CHS,), jnp.float32),
        pltpu.VMEM((16,), jnp.float32),
    ],
)
def _sc_c1(s2_hbm, src_hbm, dst_hbm, score_out, mx_out,
           ssrc, sdst, sidx, didx, sbuf, mref):
    c = lax.axis_index("c")
    s = lax.axis_index("s")
    w = c * 16 + s

    pltpu.sync_copy(s2_hbm.at[0], ssrc)
    pltpu.sync_copy(s2_hbm.at[1], sdst)
    mref[...] = jnp.full((16,), -3.0e38, jnp.float32)

    @pl.loop(0, NCHS)
    def _(k):
        e0 = w * EW + k * CHS
        pltpu.sync_copy(src_hbm.at[pl.ds(e0, CHS)], sidx)
        pltpu.sync_copy(dst_hbm.at[pl.ds(e0, CHS)], didx)

        @pl.loop(0, CHS // 16)
        def _(t):
            vs = plsc.load_gather(ssrc, [sidx[pl.ds(t * 16, 16)]])
            vd = plsc.load_gather(sdst, [didx[pl.ds(t * 16, 16)]])
            sc = vs + vd
            sc = jnp.where(sc >= 0.0, sc, 0.2 * sc)
            sbuf[pl.ds(t * 16, 16)] = sc
            mref[...] = jnp.maximum(mref[...], sc)

        pltpu.sync_copy(sbuf, score_out.at[pl.ds(e0, CHS)])

    pltpu.sync_copy(mref, mx_out.at[pl.ds(w * 16, 16)])


@functools.partial(
    pl.kernel,
    mesh=_mesh,
    compiler_params=_cp,
    out_type=jax.ShapeDtypeStruct((2, NP, D), jnp.float32),
    scratch_types=[
        pltpu.VMEM((1, CHA), jnp.int32),
        pltpu.VMEM((1, CHA), jnp.int32),
        pltpu.VMEM((CHA, D), jnp.float32),
        pltpu.VMEM((CHA,), jnp.float32),
        pltpu.VMEM((CHA,), jnp.float32),
        pltpu.VMEM((NW * 16,), jnp.float32),
        pltpu.VMEM((16,), jnp.float32),
        pltpu.VMEM_SHARED((NP, D), jnp.float32),
        pltpu.SemaphoreType.DMA,
    ],
)
def _sc_c2(score_hbm, mx_hbm, src_hbm, dst_hbm, v_hbm, out,
           sidx, didx, rows, sbuf, exbuf, mxv, mtmp, acc, sem):
    c = lax.axis_index("c")
    s = lax.axis_index("s")
    base = s * RPT
    one0 = jnp.where(lax.iota(jnp.int32, 16) == 0, 1.0, 0.0).astype(jnp.float32)

    _zero2d(rows, CHA, D)
    _zero_acc(rows, acc, base)
    plsc.subcore_barrier()

    pltpu.sync_copy(mx_hbm, mxv)
    mtmp[...] = mxv[pl.ds(0, 16)]

    @pl.loop(1, NW)
    def _(r):
        mtmp[...] = jnp.maximum(mtmp[...], mxv[pl.ds(r * 16, 16)])

    M = jnp.max(mtmp[...])

    @pl.loop(0, NCH16)
    def _(k):
        e0 = s * E16 + k * CHA
        pltpu.sync_copy(dst_hbm.at[pl.ds(e0, CHA)], didx.at[0])
        pltpu.sync_copy(score_hbm.at[pl.ds(e0, CHA)], sbuf)

        @pl.loop(0, CHA // 16)
        def _(t):
            exbuf[pl.ds(t * 16, 16)] = jnp.exp(sbuf[pl.ds(t * 16, 16)] - M)

        @pl.when(c == 0)
        def _():
            pltpu.sync_copy(src_hbm.at[pl.ds(e0, CHA)], sidx.at[0])
            pltpu.async_copy(v_hbm.at[sidx.at[0]], rows, sem).wait()

            @pl.loop(0, CHA)
            def _(r):
                bv = plsc.load_gather(exbuf, [jnp.zeros((16,), jnp.int32) + r])
                for t in range(D // 16):
                    rows[r, pl.ds(t * 16, 16)] = rows[r, pl.ds(t * 16, 16)] * bv

        @pl.when(c == 1)
        def _():
            @pl.loop(0, CHA)
            def _(r):
                bv = plsc.load_gather(exbuf, [jnp.zeros((16,), jnp.int32) + r])
                rows[r, pl.ds(0, 16)] = one0 * bv

        pltpu.sync_copy(rows, acc.at[didx.at[0]], add=True)

    plsc.subcore_barrier()
    pltpu.sync_copy(acc.at[pl.ds(base, RPT)], out.at[c, pl.ds(base, RPT)])


def _te_body(t_ref, w_ref, b_ref, o_ref):
    o_ref[...] = jnp.cos(t_ref[...] * w_ref[...] + b_ref[...])


def _dense_body(ssrc_ref, ste_ref, mem_ref, x_ref,
                wi_ref, wh_ref, bi_ref, bh_ref, asrc_ref, adst_ref, wv_ref,
                h_ref, v_ref, s2_ref):
    f32 = jnp.float32
    hi = jax.lax.Precision.HIGHEST
    ssrc = ssrc_ref[0]
    sea = ssrc_ref[1][:, :DE]
    cnt = ssrc_ref[1][:, DE]
    ste = ste_ref[0] + ste_ref[1]
    inv = (1.0 / jnp.maximum(cnt, 1.0))[:, None]
    msk = (cnt > 0.0).astype(f32)[:, None]
    mem = mem_ref[...]
    wi = wi_ref[...]
    gi = (jnp.dot(ssrc * inv, wi[0:D], preferred_element_type=f32, precision=hi)
          + jnp.dot(mem * msk, wi[D:2 * D], preferred_element_type=f32,
                    precision=hi)
          + jnp.dot(sea * inv, wi[2 * D:2 * D + DE], preferred_element_type=f32,
                    precision=hi)
          + jnp.dot(ste * inv, wi[2 * D + DE:], preferred_element_type=f32,
                    precision=hi)
          + bi_ref[...])
    gh = jnp.dot(mem, wh_ref[...], preferred_element_type=f32,
                 precision=hi) + bh_ref[...]
    r = jax.nn.sigmoid(gi[:, :D] + gh[:, :D])
    z = jax.nn.sigmoid(gi[:, D:2 * D] + gh[:, D:2 * D])
    n = jnp.tanh(gi[:, 2 * D:] + r * gh[:, 2 * D:])
    h = x_ref[...] + (1.0 - z) * n + z * mem
    h_ref[...] = h
    v_ref[...] = jnp.dot(h, wv_ref[...], preferred_element_type=f32,
                         precision=hi)
    s_src = jnp.sum(h * asrc_ref[...], axis=1)
    s_dst = jnp.sum(h * adst_ref[...], axis=1)
    s2_ref[...] = jnp.stack([s_src, s_dst], axis=0)


def _final_body(sv_ref, h_ref, o_ref):
    sv = sv_ref[0]
    den = sv_ref[1][:, 0]
    o_ref[...] = sv / (den + 1e-16)[:, None] + h_ref[...]


def kernel(x, edge_attr, edge_time, memory, W_time, b_time, W_i, W_h,
           b_i, b_h, a_src, a_dst, W_v, edge_index):
    f32 = jnp.float32
    src = edge_index[0].astype(jnp.int32)
    dst = edge_index[1].astype(jnp.int32)
    xp = jnp.pad(x.astype(f32), ((0, NP - N), (0, 0)))
    memp = jnp.pad(memory.astype(f32), ((0, NP - N), (0, 0)))

    te = pl.pallas_call(
        _te_body,
        grid=(E // BT,),
        in_specs=[
            pl.BlockSpec((BT, 1), lambda i: (i, 0)),
            pl.BlockSpec((1, D), lambda i: (0, 0)),
            pl.BlockSpec((1, D), lambda i: (0, 0)),
        ],
        out_specs=pl.BlockSpec((BT, D), lambda i: (i, 0)),
        out_shape=jax.ShapeDtypeStruct((E, D), f32),
    )(edge_time.astype(f32).reshape(E, 1), W_time.astype(f32).reshape(1, D),
      b_time.astype(f32).reshape(1, D))

    ssrc_p = _sc_a1(memp, edge_attr.astype(f32), src, dst)
    ste_p = _sc_a2(te, dst)

    h, v, s2 = pl.pallas_call(
        _dense_body,
        grid=(NP // R,),
        in_specs=[
            pl.BlockSpec((2, R, D), lambda i: (0, i, 0)),
            pl.BlockSpec((2, R, D), lambda i: (0, i, 0)),
            pl.BlockSpec((R, D), lambda i: (i, 0)),
            pl.BlockSpec((R, D), lambda i: (i, 0)),
            pl.BlockSpec((2 * D + DE + D, 3 * D), lambda i: (0, 0)),
            pl.BlockSpec((D, 3 * D), lambda i: (0, 0)),
            pl.BlockSpec((1, 3 * D), lambda i: (0, 0)),
            pl.BlockSpec((1, 3 * D), lambda i: (0, 0)),
            pl.BlockSpec((1, D), lambda i: (0, 0)),
            pl.BlockSpec((1, D), lambda i: (0, 0)),
            pl.BlockSpec((D, D), lambda i: (0, 0)),
        ],
        out_specs=[
            pl.BlockSpec((R, D), lambda i: (i, 0)),
            pl.BlockSpec((R, D), lambda i: (i, 0)),
            pl.BlockSpec((2, R), lambda i: (0, i)),
        ],
        out_shape=[
            jax.ShapeDtypeStruct((NP, D), f32),
            jax.ShapeDtypeStruct((NP, D), f32),
            jax.ShapeDtypeStruct((2, NP), f32),
        ],
    )(ssrc_p, ste_p, memp, xp,
      W_i.astype(f32), W_h.astype(f32), b_i.astype(f32).reshape(1, 3 * D),
      b_h.astype(f32).reshape(1, 3 * D), a_src.astype(f32).reshape(1, D),
      a_dst.astype(f32).reshape(1, D), W_v.astype(f32))

    score, mx = _sc_c1(s2, src, dst)
    sv_p = _sc_c2(score, mx, src, dst, v)

    outp = pl.pallas_call(
        _final_body,
        grid=(NP // R,),
        in_specs=[
            pl.BlockSpec((2, R, D), lambda i: (0, i, 0)),
            pl.BlockSpec((R, D), lambda i: (i, 0)),
        ],
        out_specs=pl.BlockSpec((R, D), lambda i: (i, 0)),
        out_shape=jax.ShapeDtypeStruct((NP, D), f32),
    )(sv_p, h)

    return outp[:N]
